# Initial kernel scaffold; baseline (speedup 1.0000x reference)
import functools

import jax
import jax.numpy as jnp
from jax import lax
from jax.experimental import pallas as pl
from jax.experimental.pallas import tpu as pltpu

DIM = 768
HID = 2048
E = 16
TPAD = 128
ROWS = 16


def _tc_body(len_ref, x_ref, scores_ref, w13_ref, w2_ref, out_ref, avg_ref):
    e = pl.program_id(0)

    def acc(i, s):
        return s + jnp.where(i < e, len_ref[i], 0)
    start = lax.fori_loop(0, E, acc, 0)
    cnt = len_ref[e]

    xe = x_ref[pl.ds(start, ROWS), :]
    inter = jnp.dot(xe, w13_ref[0], preferred_element_type=jnp.float32)
    x1 = inter[:, :HID]
    x3 = inter[:, HID:]
    h = x1 * jax.nn.sigmoid(x1) * x3
    rows = lax.broadcasted_iota(jnp.int32, (ROWS, 1), 0)
    h = jnp.where(rows < cnt, h, 0.0)
    oe = jnp.dot(h, w2_ref[0], preferred_element_type=jnp.float32)

    cur = out_ref[pl.ds(start, ROWS), :]
    out_ref[pl.ds(start, ROWS), :] = jnp.where(rows < cnt, oe, cur)

    col = lax.broadcasted_iota(jnp.int32, (1, TPAD), 1)
    in_seg = jnp.logical_and(col >= start, col < start + cnt)
    ssum = jnp.sum(jnp.where(in_seg, scores_ref[...], 0.0))
    avg = ssum / jnp.maximum(cnt, 1).astype(jnp.float32)

    @pl.when(e == 0)
    def _():
        avg_ref[...] = jnp.zeros_like(avg_ref)
    lane = lax.broadcasted_iota(jnp.int32, (1, E), 1)
    avg_ref[...] = jnp.where(lane == e, avg, avg_ref[...])


@jax.jit
def kernel(x, num_tokens_per_expert, top_scores, w13, w2):
    T = x.shape[0]
    xp = jnp.zeros((TPAD, DIM), jnp.float32).at[:T].set(x)
    sp = jnp.zeros((1, TPAD), jnp.float32).at[0, :T].set(top_scores)
    lengths = num_tokens_per_expert.astype(jnp.int32)

    out_p, avg = pl.pallas_call(
        _tc_body,
        grid=(E,),
        in_specs=[
            pl.BlockSpec(memory_space=pltpu.SMEM),
            pl.BlockSpec((TPAD, DIM), lambda e: (0, 0)),
            pl.BlockSpec((1, TPAD), lambda e: (0, 0)),
            pl.BlockSpec((1, DIM, 2 * HID), lambda e: (e, 0, 0)),
            pl.BlockSpec((1, HID, DIM), lambda e: (e, 0, 0)),
        ],
        out_specs=[
            pl.BlockSpec((TPAD, DIM), lambda e: (0, 0)),
            pl.BlockSpec((1, E), lambda e: (0, 0)),
        ],
        out_shape=[
            jax.ShapeDtypeStruct((TPAD, DIM), jnp.float32),
            jax.ShapeDtypeStruct((1, E), jnp.float32),
        ],
        compiler_params=pltpu.CompilerParams(
            dimension_semantics=("arbitrary",),
        ),
    )(lengths, xp, sp, w13, w2)

    return out_p[:T], avg[0]


# speedup vs baseline: 9.2326x; 9.2326x over previous
import functools

import jax
import jax.numpy as jnp
from jax import lax
from jax.experimental import pallas as pl
from jax.experimental.pallas import tpu as pltpu

DIM = 768
HID = 2048
E = 16
TPAD = 128
ROWS = 16


def _tc_body(len_ref, x_ref, scores_ref, w13_ref, w2_ref, out_ref, avg_ref):
    e = pl.program_id(0)

    def acc(i, s):
        return s + jnp.where(i < e, len_ref[i], 0)
    start = lax.fori_loop(0, E, acc, 0)
    cnt = len_ref[e]

    ri = lax.broadcasted_iota(jnp.int32, (ROWS, TPAD), 0)
    ti = lax.broadcasted_iota(jnp.int32, (ROWS, TPAD), 1)
    sel = jnp.logical_and(ti == start + ri, ri < cnt)
    p = sel.astype(jnp.float32)

    xe = jnp.dot(p, x_ref[...], preferred_element_type=jnp.float32)
    inter = jnp.dot(xe, w13_ref[0], preferred_element_type=jnp.float32)
    x1 = inter[:, :HID]
    x3 = inter[:, HID:]
    h = x1 * jax.nn.sigmoid(x1) * x3
    oe = jnp.dot(h, w2_ref[0], preferred_element_type=jnp.float32)

    @pl.when(e == 0)
    def _():
        out_ref[...] = jnp.zeros_like(out_ref)
    out_ref[...] += jnp.dot(p.T, oe, preferred_element_type=jnp.float32)

    col = lax.broadcasted_iota(jnp.int32, (1, TPAD), 1)
    in_seg = jnp.logical_and(col >= start, col < start + cnt)
    ssum = jnp.sum(jnp.where(in_seg, scores_ref[...], 0.0))
    avg = ssum / jnp.maximum(cnt, 1).astype(jnp.float32)

    @pl.when(e == 0)
    def _():
        avg_ref[...] = jnp.zeros_like(avg_ref)
    lane = lax.broadcasted_iota(jnp.int32, (1, E), 1)
    avg_ref[...] = jnp.where(lane == e, avg, avg_ref[...])


@jax.jit
def kernel(x, num_tokens_per_expert, top_scores, w13, w2):
    T = x.shape[0]
    xp = jnp.zeros((TPAD, DIM), jnp.float32).at[:T].set(x)
    sp = jnp.zeros((1, TPAD), jnp.float32).at[0, :T].set(top_scores)
    lengths = num_tokens_per_expert.astype(jnp.int32)

    out_p, avg = pl.pallas_call(
        _tc_body,
        grid=(E,),
        in_specs=[
            pl.BlockSpec(memory_space=pltpu.SMEM),
            pl.BlockSpec((TPAD, DIM), lambda e: (0, 0)),
            pl.BlockSpec((1, TPAD), lambda e: (0, 0)),
            pl.BlockSpec((1, DIM, 2 * HID), lambda e: (e, 0, 0)),
            pl.BlockSpec((1, HID, DIM), lambda e: (e, 0, 0)),
        ],
        out_specs=[
            pl.BlockSpec((TPAD, DIM), lambda e: (0, 0)),
            pl.BlockSpec((1, E), lambda e: (0, 0)),
        ],
        out_shape=[
            jax.ShapeDtypeStruct((TPAD, DIM), jnp.float32),
            jax.ShapeDtypeStruct((1, E), jnp.float32),
        ],
        compiler_params=pltpu.CompilerParams(
            dimension_semantics=("arbitrary",),
        ),
    )(lengths, xp, sp, w13, w2)

    return out_p[:T], avg[0]
